# Initial kernel scaffold; baseline (speedup 1.0000x reference)
import jax
import jax.numpy as jnp
from jax.experimental import pallas as pl


def kernel(x, subspace_params, centroids):
    raise NotImplementedError("write your pallas kernel here")


# speedup vs baseline: 1.9169x; 1.9169x over previous
import jax
import jax.numpy as jnp
from jax.experimental import pallas as pl
import jax.scipy.linalg as jsl

D_IN = 2048
D_OUT = 2048
CODE_DIM = 32
NUM_CENTROIDS = 512
BATCH = 4096
GROUPS_PER_TILE = 4
LANE_TILE = 128
N_TILES = D_OUT // LANE_TILE

BR = 256
BM = 1024


def _quantize_kernel(p_ref, e_ref, w_ref):
    e = e_ref[...]
    e_sq = jnp.sum(e * e, axis=1, keepdims=True)

    def tile_body(t, carry):
        p = p_ref[:, pl.ds(t * LANE_TILE, LANE_TILE)]
        score = e_sq - 2.0 * jax.lax.dot_general(
            e, p, (((1,), (1,)), ((), ())),
            preferred_element_type=jnp.float32,
            precision=jax.lax.Precision.HIGHEST,
        )
        hots = []
        for g in range(GROUPS_PER_TILE):
            sg = score[g * NUM_CENTROIDS:(g + 1) * NUM_CENTROIDS, :]
            best = jnp.min(sg, axis=0, keepdims=True)
            iota = jax.lax.broadcasted_iota(jnp.int32, sg.shape, 0)
            idx = jnp.min(jnp.where(sg <= best, iota, NUM_CENTROIDS),
                          axis=0, keepdims=True)
            hots.append((iota == idx).astype(jnp.float32))
        onehot = jnp.concatenate(hots, axis=0)
        w_ref[:, pl.ds(t * LANE_TILE, LANE_TILE)] = jax.lax.dot_general(
            onehot, e, (((0,), (0,)), ((), ())),
            preferred_element_type=jnp.float32,
        )
        return carry

    jax.lax.fori_loop(0, N_TILES, tile_body, 0)


def _matmul_kernel(x_ref, w_ref, o_ref):
    o_ref[...] = jax.lax.dot_general(
        x_ref[...], w_ref[...], (((1,), (0,)), ((), ())),
        preferred_element_type=jnp.float32,
    )


def kernel(x, subspace_params, centroids):
    p2d = subspace_params.reshape(D_IN, D_OUT)
    e_t = jsl.block_diag(*([centroids] * GROUPS_PER_TILE))
    w = pl.pallas_call(
        _quantize_kernel,
        grid=(D_IN // BR,),
        in_specs=[
            pl.BlockSpec((BR, D_OUT), lambda i: (i, 0)),
            pl.BlockSpec((GROUPS_PER_TILE * NUM_CENTROIDS, LANE_TILE),
                         lambda i: (0, 0)),
        ],
        out_specs=pl.BlockSpec((BR, D_OUT), lambda i: (i, 0)),
        out_shape=jax.ShapeDtypeStruct((D_IN, D_OUT), jnp.float32),
    )(p2d, e_t)
    out = pl.pallas_call(
        _matmul_kernel,
        grid=(BATCH // BM,),
        in_specs=[
            pl.BlockSpec((BM, D_IN), lambda i: (i, 0)),
            pl.BlockSpec((D_IN, D_OUT), lambda i: (0, 0)),
        ],
        out_specs=pl.BlockSpec((BM, D_OUT), lambda i: (i, 0)),
        out_shape=jax.ShapeDtypeStruct((BATCH, D_OUT), jnp.float32),
    )(x, w)
    return out


# speedup vs baseline: 3.3019x; 1.7225x over previous
import jax
import jax.numpy as jnp
from jax.experimental import pallas as pl
import jax.scipy.linalg as jsl

D_IN = 2048
D_OUT = 2048
CODE_DIM = 32
NUM_CENTROIDS = 512
BATCH = 4096
GROUPS_PER_TILE = 4
LANE_TILE = 128
N_TILES = D_OUT // LANE_TILE

BR = 256
BM = 1024


def _quantize_kernel(p_ref, c_ref, e_ref, w_ref):
    c = c_ref[...]
    e = e_ref[...]
    c_sq = jnp.sum(c * c, axis=1, keepdims=True)

    def tile_body(t, carry):
        p = p_ref[:, pl.ds(t * LANE_TILE, LANE_TILE)]
        hots = []
        for g in range(GROUPS_PER_TILE):
            pg = p[:, g * CODE_DIM:(g + 1) * CODE_DIM]
            sg = c_sq - 2.0 * jax.lax.dot_general(
                c, pg, (((1,), (1,)), ((), ())),
                preferred_element_type=jnp.float32,
            )
            best = jnp.min(sg, axis=0, keepdims=True)
            iota = jax.lax.broadcasted_iota(jnp.int32, sg.shape, 0)
            idx = jnp.min(jnp.where(sg <= best, iota, NUM_CENTROIDS),
                          axis=0, keepdims=True)
            hots.append((iota == idx).astype(jnp.float32))
        onehot = jnp.concatenate(hots, axis=0)
        w_ref[:, pl.ds(t * LANE_TILE, LANE_TILE)] = jax.lax.dot_general(
            onehot, e, (((0,), (0,)), ((), ())),
            preferred_element_type=jnp.float32,
        )
        return carry

    jax.lax.fori_loop(0, N_TILES, tile_body, 0)


def _matmul_kernel(x_ref, w_ref, o_ref):
    o_ref[...] = jax.lax.dot_general(
        x_ref[...], w_ref[...], (((1,), (0,)), ((), ())),
        preferred_element_type=jnp.float32,
    )


def kernel(x, subspace_params, centroids):
    p2d = subspace_params.reshape(D_IN, D_OUT)
    e_t = jsl.block_diag(*([centroids] * GROUPS_PER_TILE))
    w = pl.pallas_call(
        _quantize_kernel,
        grid=(D_IN // BR,),
        in_specs=[
            pl.BlockSpec((BR, D_OUT), lambda i: (i, 0)),
            pl.BlockSpec((NUM_CENTROIDS, CODE_DIM), lambda i: (0, 0)),
            pl.BlockSpec((GROUPS_PER_TILE * NUM_CENTROIDS, LANE_TILE),
                         lambda i: (0, 0)),
        ],
        out_specs=pl.BlockSpec((BR, D_OUT), lambda i: (i, 0)),
        out_shape=jax.ShapeDtypeStruct((D_IN, D_OUT), jnp.float32),
    )(p2d, centroids, e_t)
    out = pl.pallas_call(
        _matmul_kernel,
        grid=(BATCH // BM,),
        in_specs=[
            pl.BlockSpec((BM, D_IN), lambda i: (i, 0)),
            pl.BlockSpec((D_IN, D_OUT), lambda i: (0, 0)),
        ],
        out_specs=pl.BlockSpec((BM, D_OUT), lambda i: (i, 0)),
        out_shape=jax.ShapeDtypeStruct((BATCH, D_OUT), jnp.float32),
    )(x, w)
    return out
